# baseline (speedup 1.0000x reference)
import functools

import jax
import jax.numpy as jnp
from jax import lax
from jax.experimental import pallas as pl
from jax.experimental.pallas import tpu as pltpu
from jax.experimental.pallas import tpu_sc as plsc

_B = 16384
_D = 64
_NC = 2
_NS = 16
_NW = _NC * _NS
_BPW = _B // _NW

_MM_BLOCK = 2048


@functools.partial(
    pl.kernel,
    out_type=[
        jax.ShapeDtypeStruct((_B, _D), jnp.float32),
        jax.ShapeDtypeStruct((_D, _D), jnp.float32),
    ],
    mesh=plsc.VectorSubcoreMesh(core_axis_name="c", subcore_axis_name="s"),
    compiler_params=pltpu.CompilerParams(use_tc_tiling_on_sc=False),
    scratch_types=[
        pltpu.VMEM((_BPW,), jnp.int32),
        pltpu.VMEM((_BPW, _D), jnp.float32),
        pltpu.VMEM((_D,), jnp.int32),
        pltpu.VMEM((_D, _D), jnp.float32),
        pltpu.SemaphoreType.DMA,
    ],
)
def _sc_gather(user_idx_hbm, item_idx_hbm, eu_tab, ei_tab,
               eu_out, ei_out, idx_v, rows_v, iidx_v, irows_v, sem):
    wid = lax.axis_index("s") * _NC + lax.axis_index("c")
    base = wid * _BPW
    pltpu.sync_copy(user_idx_hbm.at[pl.ds(base, _BPW)], idx_v)
    pltpu.async_copy(eu_tab.at[idx_v], rows_v, sem).wait()
    pltpu.sync_copy(rows_v, eu_out.at[pl.ds(base, _BPW)])

    @pl.when(wid == 0)
    def _():
        pltpu.sync_copy(item_idx_hbm, iidx_v)
        pltpu.async_copy(ei_tab.at[iidx_v], irows_v, sem).wait()
        pltpu.sync_copy(irows_v, ei_out)


def _mm_body(eu_ref, ei_ref, out_ref):
    out_ref[...] = jnp.dot(eu_ref[...], ei_ref[...],
                           preferred_element_type=jnp.float32)


_mm = pl.pallas_call(
    _mm_body,
    grid=(_B // _MM_BLOCK,),
    in_specs=[
        pl.BlockSpec((_MM_BLOCK, _D), lambda i: (i, 0)),
        pl.BlockSpec((_D, _D), lambda i: (0, 0)),
    ],
    out_specs=pl.BlockSpec((_MM_BLOCK, _D), lambda i: (i, 0)),
    out_shape=jax.ShapeDtypeStruct((_B, _D), jnp.float32),
)


def kernel(user_idx, item_idx, embed_user, embed_item):
    user_idx = user_idx.astype(jnp.int32)
    item_idx = item_idx.astype(jnp.int32)
    eu, ei = _sc_gather(user_idx, item_idx, embed_user, embed_item)
    return _mm(eu, ei)


# speedup vs baseline: 1.4356x; 1.4356x over previous
import functools

import jax
import jax.numpy as jnp
from jax import lax
from jax.experimental import pallas as pl
from jax.experimental.pallas import tpu as pltpu
from jax.experimental.pallas import tpu_sc as plsc

_B = 16384
_D = 64
_N = 100000
_NC = 2
_NS = 16
_NW = _NC * _NS
_BPW = _B // _NW
_PAD = 128
_PB = 2048
_PGRID = -(-_N // _PB)


def _item_body(idx_ref, eit_ref, h_ref):
    k = pl.program_id(0)
    lane = idx_ref[k] % _PAD
    onehot = (lax.broadcasted_iota(jnp.int32, (_PAD, _D), 0) == lane)
    col = jnp.dot(eit_ref[...], onehot.astype(jnp.float32),
                  preferred_element_type=jnp.float32)
    sel = lax.broadcasted_iota(jnp.int32, (_D, _D), 1) == k
    h_ref[...] = jnp.where(sel, col, h_ref[...])


_item_gather = pl.pallas_call(
    _item_body,
    grid_spec=pltpu.PrefetchScalarGridSpec(
        num_scalar_prefetch=1,
        grid=(_D,),
        in_specs=[pl.BlockSpec((_D, _PAD), lambda k, idx: (0, idx[k] // _PAD))],
        out_specs=pl.BlockSpec((_D, _D), lambda k, idx: (0, 0)),
    ),
    out_shape=jax.ShapeDtypeStruct((_D, _D), jnp.float32),
)


def _p_body(eut_ref, h_ref, p_ref):
    p = lax.dot_general(
        eut_ref[...], h_ref[...], (((0,), (1,)), ((), ())),
        preferred_element_type=jnp.float32,
    )
    p_ref[:, :_D] = p


_p_matmul = pl.pallas_call(
    _p_body,
    grid=(_PGRID,),
    in_specs=[
        pl.BlockSpec((_D, _PB), lambda i: (0, i)),
        pl.BlockSpec((_D, _D), lambda i: (0, 0)),
    ],
    out_specs=pl.BlockSpec((_PB, _PAD), lambda i: (i, 0)),
    out_shape=jax.ShapeDtypeStruct((_N, _PAD), jnp.float32),
)


@functools.partial(
    pl.kernel,
    out_type=jax.ShapeDtypeStruct((_B, _PAD), jnp.float32),
    mesh=plsc.VectorSubcoreMesh(core_axis_name="c", subcore_axis_name="s"),
    scratch_types=[
        pltpu.VMEM((_BPW,), jnp.int32),
        pltpu.VMEM((_BPW, _PAD), jnp.float32),
        pltpu.SemaphoreType.DMA,
    ],
)
def _sc_gather(user_idx_hbm, p_hbm, out_hbm, idx_v, rows_v, sem):
    wid = lax.axis_index("s") * _NC + lax.axis_index("c")
    base = wid * _BPW
    pltpu.sync_copy(user_idx_hbm.at[pl.ds(base, _BPW)], idx_v)
    pltpu.async_copy(p_hbm.at[idx_v], rows_v, sem).wait()
    pltpu.sync_copy(rows_v, out_hbm.at[pl.ds(base, _BPW)])


def kernel(user_idx, item_idx, embed_user, embed_item):
    user_idx = user_idx.astype(jnp.int32)
    item_idx = item_idx.astype(jnp.int32)
    eu_t = embed_user.T
    ei_t = embed_item.T
    h = _item_gather(item_idx, ei_t)
    p = _p_matmul(eu_t, h)
    outp = _sc_gather(user_idx, p)
    return outp[:, :_D]


# speedup vs baseline: 2.0588x; 1.4341x over previous
import functools

import jax
import jax.numpy as jnp
from jax import lax
from jax.experimental import pallas as pl
from jax.experimental.pallas import tpu as pltpu
from jax.experimental.pallas import tpu_sc as plsc

_B = 16384
_D = 64
_N = 100000
_NC = 2
_NS = 16
_NW = _NC * _NS
_BPW = _B // _NW
_PAD = 128
_PB = 4096
_PGRID = -(-_N // _PB)


_IPG = 4


def _item_body(idx_ref, *refs):
    h_ref = refs[-1]
    g = pl.program_id(0)
    acc = jnp.zeros((_D, _D), jnp.float32)
    for j in range(_IPG):
        k = g * _IPG + j
        lane = idx_ref[k] % _PAD
        onehot = (lax.broadcasted_iota(jnp.int32, (_PAD, _D), 0) == lane)
        col = jnp.dot(refs[j][...], onehot.astype(jnp.float32),
                      preferred_element_type=jnp.float32)
        sel = lax.broadcasted_iota(jnp.int32, (_D, _D), 1) == k
        acc = jnp.where(sel, col, acc)
    sel_g = (lax.broadcasted_iota(jnp.int32, (_D, _D), 1) // _IPG) == g
    h_ref[...] = jnp.where(sel_g, acc, h_ref[...])


_item_gather = pl.pallas_call(
    _item_body,
    grid_spec=pltpu.PrefetchScalarGridSpec(
        num_scalar_prefetch=1,
        grid=(_D // _IPG,),
        in_specs=[
            pl.BlockSpec((_D, _PAD),
                         functools.partial(
                             lambda j, g, idx: (0, idx[g * _IPG + j] // _PAD), j))
            for j in range(_IPG)
        ],
        out_specs=pl.BlockSpec((_D, _D), lambda g, idx: (0, 0)),
    ),
    out_shape=jax.ShapeDtypeStruct((_D, _D), jnp.float32),
)


def _p_body(eut_ref, h_ref, p_ref):
    p = lax.dot_general(
        eut_ref[...], h_ref[...], (((0,), (1,)), ((), ())),
        preferred_element_type=jnp.float32,
    )
    p_ref[:, :_D] = p


_p_matmul = pl.pallas_call(
    _p_body,
    grid=(_PGRID,),
    in_specs=[
        pl.BlockSpec((_D, _PB), lambda i: (0, i)),
        pl.BlockSpec((_D, _D), lambda i: (0, 0)),
    ],
    out_specs=pl.BlockSpec((_PB, _PAD), lambda i: (i, 0)),
    out_shape=jax.ShapeDtypeStruct((_N, _PAD), jnp.float32),
)


@functools.partial(
    pl.kernel,
    out_type=jax.ShapeDtypeStruct((_B, _PAD), jnp.float32),
    mesh=plsc.VectorSubcoreMesh(core_axis_name="c", subcore_axis_name="s"),
    scratch_types=[
        pltpu.VMEM((_BPW,), jnp.int32),
        pltpu.VMEM((_BPW, _PAD), jnp.float32),
        pltpu.SemaphoreType.DMA,
    ],
)
def _sc_gather(user_idx_hbm, p_hbm, out_hbm, idx_v, rows_v, sem):
    wid = lax.axis_index("s") * _NC + lax.axis_index("c")
    base = wid * _BPW
    pltpu.sync_copy(user_idx_hbm.at[pl.ds(base, _BPW)], idx_v)
    pltpu.async_copy(p_hbm.at[idx_v], rows_v, sem).wait()
    pltpu.sync_copy(rows_v, out_hbm.at[pl.ds(base, _BPW)])


def kernel(user_idx, item_idx, embed_user, embed_item):
    user_idx = user_idx.astype(jnp.int32)
    item_idx = item_idx.astype(jnp.int32)
    eu_t = embed_user.T
    ei_t = embed_item.T
    h = _item_gather(item_idx, ei_t, ei_t, ei_t, ei_t)
    p = _p_matmul(eu_t, h)
    outp = _sc_gather(user_idx, p)
    return outp[:, :_D]


# speedup vs baseline: 2.2646x; 1.1000x over previous
import functools

import jax
import jax.numpy as jnp
from jax import lax
from jax.experimental import pallas as pl
from jax.experimental.pallas import tpu as pltpu
from jax.experimental.pallas import tpu_sc as plsc

_B = 16384
_D = 64
_N = 100000
_NC = 2
_NS = 16
_NW = _NC * _NS
_BPW = _B // _NW
_PAD = 128
_PB = 4096
_PGRID = -(-_N // _PB)


def _item_body(idx_ref, *refs):
    h_ref = refs[-1]
    acc = jnp.zeros((_D, _D), jnp.float32)
    for k in range(_D):
        lane = idx_ref[k] % _PAD
        onehot = (lax.broadcasted_iota(jnp.int32, (_PAD, _D), 0) == lane)
        col = jnp.dot(refs[k][...], onehot.astype(jnp.float32),
                      preferred_element_type=jnp.float32)
        sel = lax.broadcasted_iota(jnp.int32, (_D, _D), 1) == k
        acc = jnp.where(sel, col, acc)
    h_ref[...] = acc


_item_gather = pl.pallas_call(
    _item_body,
    grid_spec=pltpu.PrefetchScalarGridSpec(
        num_scalar_prefetch=1,
        grid=(1,),
        in_specs=[
            pl.BlockSpec((_D, _PAD),
                         functools.partial(
                             lambda k, g, idx: (0, idx[k] // _PAD), k))
            for k in range(_D)
        ],
        out_specs=pl.BlockSpec((_D, _D), lambda g, idx: (0, 0)),
    ),
    out_shape=jax.ShapeDtypeStruct((_D, _D), jnp.float32),
)


def _p_body(eut_ref, h_ref, p_ref):
    p = lax.dot_general(
        eut_ref[...], h_ref[...], (((0,), (1,)), ((), ())),
        preferred_element_type=jnp.float32,
    )
    p_ref[:, :_D] = p


_p_matmul = pl.pallas_call(
    _p_body,
    grid=(_PGRID,),
    in_specs=[
        pl.BlockSpec((_D, _PB), lambda i: (0, i)),
        pl.BlockSpec((_D, _D), lambda i: (0, 0)),
    ],
    out_specs=pl.BlockSpec((_PB, _PAD), lambda i: (i, 0)),
    out_shape=jax.ShapeDtypeStruct((_N, _PAD), jnp.float32),
)


@functools.partial(
    pl.kernel,
    out_type=jax.ShapeDtypeStruct((_B, _PAD), jnp.float32),
    mesh=plsc.VectorSubcoreMesh(core_axis_name="c", subcore_axis_name="s"),
    scratch_types=[
        pltpu.VMEM((_BPW,), jnp.int32),
        pltpu.VMEM((_BPW, _PAD), jnp.float32),
        pltpu.SemaphoreType.DMA,
    ],
)
def _sc_gather(user_idx_hbm, p_hbm, out_hbm, idx_v, rows_v, sem):
    wid = lax.axis_index("s") * _NC + lax.axis_index("c")
    base = wid * _BPW
    pltpu.sync_copy(user_idx_hbm.at[pl.ds(base, _BPW)], idx_v)
    pltpu.async_copy(p_hbm.at[idx_v], rows_v, sem).wait()
    pltpu.sync_copy(rows_v, out_hbm.at[pl.ds(base, _BPW)])


def kernel(user_idx, item_idx, embed_user, embed_item):
    user_idx = user_idx.astype(jnp.int32)
    item_idx = item_idx.astype(jnp.int32)
    eu_t = embed_user.T
    ei_t = embed_item.T
    h = _item_gather(item_idx, *([ei_t] * _D))
    p = _p_matmul(eu_t, h)
    outp = _sc_gather(user_idx, p)
    return outp[:, :_D]


# speedup vs baseline: 2.5777x; 1.1382x over previous
import functools

import jax
import jax.numpy as jnp
from jax import lax
from jax.experimental import pallas as pl
from jax.experimental.pallas import tpu as pltpu
from jax.experimental.pallas import tpu_sc as plsc

_B = 16384
_D = 64
_N = 100000
_NC = 2
_NS = 16
_NW = _NC * _NS
_BPW = _B // _NW
_PAD = 128
_PB = 8192
_PGRID = -(-_N // _PB)


def _item_body(idx_ref, *refs):
    h_ref = refs[-1]
    acc = jnp.zeros((_D, _D), jnp.float32)
    for k in range(_D):
        lane = idx_ref[k] % _PAD
        onehot = (lax.broadcasted_iota(jnp.int32, (_PAD, _D), 0) == lane)
        col = jnp.dot(refs[k][...], onehot.astype(jnp.float32),
                      preferred_element_type=jnp.float32)
        sel = lax.broadcasted_iota(jnp.int32, (_D, _D), 1) == k
        acc = jnp.where(sel, col, acc)
    h_ref[...] = acc


_item_gather = pl.pallas_call(
    _item_body,
    grid_spec=pltpu.PrefetchScalarGridSpec(
        num_scalar_prefetch=1,
        grid=(1,),
        in_specs=[
            pl.BlockSpec((_D, _PAD),
                         functools.partial(
                             lambda k, g, idx: (0, idx[k] // _PAD), k))
            for k in range(_D)
        ],
        out_specs=pl.BlockSpec((_D, _D), lambda g, idx: (0, 0)),
    ),
    out_shape=jax.ShapeDtypeStruct((_D, _D), jnp.float32),
)


def _p_body(eut_ref, h_ref, p_ref):
    p = lax.dot_general(
        eut_ref[...], h_ref[...], (((0,), (1,)), ((), ())),
        preferred_element_type=jnp.float32,
    )
    p_ref[:, :_D] = p


_p_matmul = pl.pallas_call(
    _p_body,
    grid=(_PGRID,),
    in_specs=[
        pl.BlockSpec((_D, _PB), lambda i: (0, i)),
        pl.BlockSpec((_D, _D), lambda i: (0, 0)),
    ],
    out_specs=pl.BlockSpec((_PB, _PAD), lambda i: (i, 0)),
    out_shape=jax.ShapeDtypeStruct((_N, _PAD), jnp.float32),
)


@functools.partial(
    pl.kernel,
    out_type=jax.ShapeDtypeStruct((_B, _PAD), jnp.float32),
    mesh=plsc.VectorSubcoreMesh(core_axis_name="c", subcore_axis_name="s"),
    scratch_types=[
        pltpu.VMEM((_BPW,), jnp.int32),
        pltpu.VMEM((_BPW, _PAD), jnp.float32),
        pltpu.SemaphoreType.DMA,
    ],
)
def _sc_gather(user_idx_hbm, p_hbm, out_hbm, idx_v, rows_v, sem):
    wid = lax.axis_index("s") * _NC + lax.axis_index("c")
    base = wid * _BPW
    pltpu.sync_copy(user_idx_hbm.at[pl.ds(base, _BPW)], idx_v)
    pltpu.async_copy(p_hbm.at[idx_v], rows_v, sem).wait()
    pltpu.sync_copy(rows_v, out_hbm.at[pl.ds(base, _BPW)])


def kernel(user_idx, item_idx, embed_user, embed_item):
    user_idx = user_idx.astype(jnp.int32)
    item_idx = item_idx.astype(jnp.int32)
    eu_t = embed_user.T
    ei_t = embed_item.T
    h = _item_gather(item_idx, *([ei_t] * _D))
    p = _p_matmul(eu_t, h)
    outp = _sc_gather(user_idx, p)
    return outp[:, :_D]


# speedup vs baseline: 2.6579x; 1.0311x over previous
import functools

import jax
import jax.numpy as jnp
from jax import lax
from jax.experimental import pallas as pl
from jax.experimental.pallas import tpu as pltpu
from jax.experimental.pallas import tpu_sc as plsc

_B = 16384
_D = 64
_N = 100000
_NC = 2
_NS = 16
_NW = _NC * _NS
_BPW = _B // _NW
_PAD = 128
_PB = 16384
_PGRID = -(-_N // _PB)


def _item_body(idx_ref, *refs):
    h_ref = refs[-1]
    acc = jnp.zeros((_D, _D), jnp.float32)
    for k in range(_D):
        lane = idx_ref[k] % _PAD
        onehot = (lax.broadcasted_iota(jnp.int32, (_PAD, _D), 0) == lane)
        col = jnp.dot(refs[k][...], onehot.astype(jnp.float32),
                      preferred_element_type=jnp.float32)
        sel = lax.broadcasted_iota(jnp.int32, (_D, _D), 1) == k
        acc = jnp.where(sel, col, acc)
    h_ref[...] = acc


_item_gather = pl.pallas_call(
    _item_body,
    grid_spec=pltpu.PrefetchScalarGridSpec(
        num_scalar_prefetch=1,
        grid=(1,),
        in_specs=[
            pl.BlockSpec((_D, _PAD),
                         functools.partial(
                             lambda k, g, idx: (0, idx[k] // _PAD), k))
            for k in range(_D)
        ],
        out_specs=pl.BlockSpec((_D, _D), lambda g, idx: (0, 0)),
    ),
    out_shape=jax.ShapeDtypeStruct((_D, _D), jnp.float32),
)


def _p_body(eut_ref, h_ref, p_ref):
    p = lax.dot_general(
        eut_ref[...], h_ref[...], (((0,), (1,)), ((), ())),
        preferred_element_type=jnp.float32,
    )
    p_ref[:, :_D] = p


_p_matmul = pl.pallas_call(
    _p_body,
    grid=(_PGRID,),
    in_specs=[
        pl.BlockSpec((_D, _PB), lambda i: (0, i)),
        pl.BlockSpec((_D, _D), lambda i: (0, 0)),
    ],
    out_specs=pl.BlockSpec((_PB, _PAD), lambda i: (i, 0)),
    out_shape=jax.ShapeDtypeStruct((_N, _PAD), jnp.float32),
)


@functools.partial(
    pl.kernel,
    out_type=jax.ShapeDtypeStruct((_B, _PAD), jnp.float32),
    mesh=plsc.VectorSubcoreMesh(core_axis_name="c", subcore_axis_name="s"),
    scratch_types=[
        pltpu.VMEM((_BPW,), jnp.int32),
        pltpu.VMEM((_BPW, _PAD), jnp.float32),
        pltpu.SemaphoreType.DMA,
    ],
)
def _sc_gather(user_idx_hbm, p_hbm, out_hbm, idx_v, rows_v, sem):
    wid = lax.axis_index("s") * _NC + lax.axis_index("c")
    base = wid * _BPW
    pltpu.sync_copy(user_idx_hbm.at[pl.ds(base, _BPW)], idx_v)
    pltpu.async_copy(p_hbm.at[idx_v], rows_v, sem).wait()
    pltpu.sync_copy(rows_v, out_hbm.at[pl.ds(base, _BPW)])


def kernel(user_idx, item_idx, embed_user, embed_item):
    user_idx = user_idx.astype(jnp.int32)
    item_idx = item_idx.astype(jnp.int32)
    eu_t = embed_user.T
    ei_t = embed_item.T
    h = _item_gather(item_idx, *([ei_t] * _D))
    p = _p_matmul(eu_t, h)
    outp = _sc_gather(user_idx, p)
    return outp[:, :_D]
